# baseline (speedup 1.0000x reference)
import jax
import jax.numpy as jnp
from jax import lax
from jax.experimental import pallas as pl

_T, _B, _N, _FEAT, _R, _EMB = 2, 8, 300, 32, 3, 16


def _dot_t(a, b):
    return lax.dot_general(a, b, (((0,), (0,)), ((), ())),
                           preferred_element_type=jnp.float32)


_NP = _N + 4


def _gnn_kernel(xt_ref, adj_ref, embw_ref, embb_ref,
                w1_ref, r1_ref, b1_ref, w2_ref, r2_ref, b2_ref, out_ref):
    mm = lambda a, c: jnp.dot(a, c, preferred_element_type=jnp.float32)

    lmask = (lax.broadcasted_iota(jnp.int32, (1, _NP), 1) < _N
             ).astype(jnp.float32)
    widen = lambda v: jnp.pad(v, ((0, 0), (0, _NP - _N))) * lmask

    hs = [widen(_dot_t(embw_ref[...], xt_ref[0, b]) + embb_ref[...])
          for b in range(_B)]
    o1s = [_dot_t(r1_ref[...], h)[:, :_N] + b1_ref[...] for h in hs]

    ones_row = lmask[:, :_N]
    vts, invs = [], []
    for r in range(_R):
        vt = jnp.swapaxes(adj_ref[0, :, r, :, :], 0, 1).astype(jnp.float32)
        vts.append(vt)
        inv = [1.0 / jnp.maximum(mm(ones_row, vt[b]), 1.0)
               for b in range(_B)]
        invs.append(inv)
        for b in range(_B):
            hw = _dot_t(w1_ref[r], hs[b])
            o1s[b] = o1s[b] + mm(hw[:, :_N], vt[b]) * inv[b]

    for b in range(_B):
        h1 = widen(jnp.maximum(o1s[b], 0.0))
        o = _dot_t(r2_ref[...], h1)[:, :_N] + b2_ref[...]
        for r in range(_R):
            hw = _dot_t(w2_ref[r], h1)
            o = o + mm(hw[:, :_N], vts[r][b]) * invs[r][b]
        out_ref[0, b] = jnp.maximum(o, 0.0)


def kernel(unary_tensor, binary_tensor, emb_W, emb_b, W1, root1, b1, W2, root2, b2):
    xt = unary_tensor.astype(jnp.float32).transpose(0, 1, 3, 2)
    adj = binary_tensor.transpose(0, 2, 4, 1, 3)
    full = lambda *s: pl.BlockSpec(s, lambda t: (0,) * len(s))
    out = pl.pallas_call(
        _gnn_kernel,
        grid=(_T,),
        in_specs=[
            pl.BlockSpec((1, _B, _FEAT, _N), lambda t: (t, 0, 0, 0)),
            pl.BlockSpec((1, _N, _R, _B, _N), lambda t: (t, 0, 0, 0, 0)),
            full(_FEAT, _EMB),
            full(_EMB, 1),
            full(_R, _EMB, _EMB),
            full(_EMB, _EMB),
            full(_EMB, 1),
            full(_R, _EMB, _EMB),
            full(_EMB, _EMB),
            full(_EMB, 1),
        ],
        out_specs=pl.BlockSpec((1, _B, _EMB, _N), lambda t: (t, 0, 0, 0)),
        out_shape=jax.ShapeDtypeStruct((_T, _B, _EMB, _N), jnp.float32),
    )(xt, adj, emb_W, emb_b.reshape(_EMB, 1), W1, root1, b1.reshape(_EMB, 1),
      W2, root2, b2.reshape(_EMB, 1))
    return out.transpose(0, 1, 3, 2).reshape(_T * _B, _N * _EMB)
